# baseline (speedup 1.0000x reference)
import functools

import jax
import jax.numpy as jnp
from jax import lax
from jax.experimental import pallas as pl
from jax.experimental.pallas import tpu as pltpu
from jax.experimental.pallas import tpu_sc as plsc

B = 16384
EMB = 32
SEQ = 20
NW = 32
RPW = B // NW
CH = 128
NCH = RPW // CH
NG = RPW // 16


def _sc_body(idx_hbm, tok_hbm, idtab_hbm, txttab_hbm, out_hbm,
             tokrm_v, toks_v, ididx_v, idrows_v, acc_v, a_v, b_v, row0_v,
             out_v, sem_id, sem_tx):
  c = lax.axis_index("c")
  s = lax.axis_index("s")
  wid = s * 2 + c
  base = wid * RPW

  def _zero(r, _):
    z = jnp.zeros((16,), jnp.float32)
    acc_v[r, pl.ds(0, 16)] = z
    acc_v[r, pl.ds(16, 16)] = z
    return _
  lax.fori_loop(0, RPW, _zero, None)

  pltpu.sync_copy(tok_hbm.at[pl.ds(base, RPW)], tokrm_v)
  pltpu.sync_copy(idx_hbm.at[pl.ds(base, RPW)], ididx_v)
  pltpu.sync_copy(txttab_hbm.at[pl.ds(0, 1)], row0_v)

  id_copies = []
  for k in range(NCH):
    cp = pltpu.make_async_copy(
        idtab_hbm.at[ididx_v.at[pl.ds(k * CH, CH)]],
        idrows_v.at[pl.ds(k * CH, CH)],
        sem_id)
    cp.start()
    id_copies.append(cp)

  lanes = lax.iota(jnp.int32, 16)
  one = jnp.ones((16,), jnp.float32)
  zero = jnp.zeros((16,), jnp.float32)

  def _tr(g, _):
    r16 = pl.multiple_of(g * 16, 16)
    rows = r16 + lanes
    cnt = jnp.zeros((16,), jnp.float32)
    for j in range(SEQ):
      v = plsc.load_gather(tokrm_v, [rows, jnp.full((16,), j, jnp.int32)])
      cnt = cnt + jnp.where(v != 0, one, zero)
      toks_v[j, pl.ds(r16, 16)] = v
    denom = jnp.maximum(cnt, 1.0)
    a_v[pl.ds(r16, 16)] = 1.0 / denom
    b_v[pl.ds(r16, 16)] = (float(SEQ) - cnt) / denom
    return _
  lax.fori_loop(0, NG, _tr, None)

  tx_copies = []
  for k in range(NCH):
    for j in range(SEQ):
      cp = pltpu.make_async_copy(
          txttab_hbm.at[toks_v.at[j, pl.ds(k * CH, CH)]],
          acc_v.at[pl.ds(k * CH, CH)],
          sem_tx)
      cp.start(add=True)
      tx_copies.append(cp)

  for cp in tx_copies:
    cp.wait()
  for cp in id_copies:
    cp.wait()

  r0_lo = row0_v[0, pl.ds(0, 16)]
  r0_hi = row0_v[0, pl.ds(16, 16)]

  def _scale(g, _):
    r16 = pl.multiple_of(g * 16, 16)
    avec = a_v[pl.ds(r16, 16)]
    bvec = b_v[pl.ds(r16, 16)]
    for l in range(16):
      a = avec[l]
      b = bvec[l]
      r = r16 + l
      out_v[r, pl.ds(0, 16)] = idrows_v[r, pl.ds(0, 16)]
      out_v[r, pl.ds(16, 16)] = idrows_v[r, pl.ds(16, 16)]
      lo = acc_v[r, pl.ds(0, 16)]
      hi = acc_v[r, pl.ds(16, 16)]
      out_v[r, pl.ds(32, 16)] = lo * a - r0_lo * b
      out_v[r, pl.ds(48, 16)] = hi * a - r0_hi * b
    return _
  lax.fori_loop(0, NG, _scale, None)

  pltpu.sync_copy(out_v, out_hbm.at[pl.ds(base, RPW)])


@functools.partial(jax.jit, static_argnums=())
def _run(jnrl_id_idx, text_token_ids, id_table, text_table):
  mesh = plsc.VectorSubcoreMesh(core_axis_name="c", subcore_axis_name="s")
  f = pl.kernel(
      _sc_body,
      out_type=jax.ShapeDtypeStruct((B, 2 * EMB), jnp.float32),
      mesh=mesh,
      compiler_params=pltpu.CompilerParams(
          use_tc_tiling_on_sc=False, needs_layout_passes=False),
      scratch_types=[
          pltpu.VMEM((RPW, SEQ), jnp.int32),
          pltpu.VMEM((SEQ, RPW), jnp.int32),
          pltpu.VMEM((RPW,), jnp.int32),
          pltpu.VMEM((RPW, EMB), jnp.float32),
          pltpu.VMEM((RPW, EMB), jnp.float32),
          pltpu.VMEM((RPW,), jnp.float32),
          pltpu.VMEM((RPW,), jnp.float32),
          pltpu.VMEM((1, EMB), jnp.float32),
          pltpu.VMEM((RPW, 2 * EMB), jnp.float32),
          pltpu.SemaphoreType.DMA,
          pltpu.SemaphoreType.DMA,
      ],
  )
  return f(jnrl_id_idx, text_token_ids, id_table, text_table)


def kernel(jnrl_id_idx, text_token_ids, id_table, text_table):
  return _run(jnrl_id_idx, text_token_ids, id_table, text_table)


# speedup vs baseline: 1.3489x; 1.3489x over previous
import functools

import jax
import jax.numpy as jnp
from jax import lax
from jax.experimental import pallas as pl
from jax.experimental.pallas import tpu as pltpu
from jax.experimental.pallas import tpu_sc as plsc

B = 16384
ID_V = 100001
EMB = 32
SEQ = 20
NW = 32
RPW = B // NW
CH = 128
NCH = RPW // CH
NG = RPW // 16


def _sc_body(idx_hbm, tokT_hbm, idtabT_hbm, txttab_hbm, outT_hbm,
             toks_v, ididx_v, gidx_v, idcols_v, acc_v, txtT_v, a_v, b_v,
             row0_v, sem_id, sem_tx):
  c = lax.axis_index("c")
  s = lax.axis_index("s")
  wid = s * 2 + c
  base = wid * RPW

  def _zero(r, _):
    z = jnp.zeros((16,), jnp.float32)
    acc_v[r, pl.ds(0, 16)] = z
    acc_v[r, pl.ds(16, 16)] = z
    return _
  lax.fori_loop(0, RPW, _zero, None)

  pltpu.sync_copy(tokT_hbm.at[:, pl.ds(base, RPW)], toks_v)
  pltpu.sync_copy(idx_hbm.at[pl.ds(base, RPW)], ididx_v)
  pltpu.sync_copy(txttab_hbm.at[pl.ds(0, 1)], row0_v)

  tx_copies = []
  for k in range(NCH):
    for j in range(SEQ):
      cp = pltpu.make_async_copy(
          txttab_hbm.at[toks_v.at[j, pl.ds(k * CH, CH)]],
          acc_v.at[pl.ds(k * CH, CH)],
          sem_tx)
      cp.start(add=True)
      tx_copies.append(cp)

  def _gidx(g, _):
    r16 = pl.multiple_of(g * 16, 16)
    iv = ididx_v[pl.ds(r16, 16)]
    for cc in range(EMB):
      gidx_v[cc, pl.ds(r16, 16)] = iv + (cc * ID_V)
    return _
  lax.fori_loop(0, NG, _gidx, None)

  id_copies = []
  for cc in range(EMB):
    for k in range(NCH):
      cp = pltpu.make_async_copy(
          idtabT_hbm.at[gidx_v.at[cc, pl.ds(k * CH, CH)]],
          idcols_v.at[cc, pl.ds(k * CH, CH)],
          sem_id)
      cp.start()
      id_copies.append(cp)

  one = jnp.ones((16,), jnp.float32)
  zero = jnp.zeros((16,), jnp.float32)

  def _count(g, _):
    r16 = pl.multiple_of(g * 16, 16)
    cnt = jnp.zeros((16,), jnp.float32)
    for j in range(SEQ):
      v = toks_v[j, pl.ds(r16, 16)]
      cnt = cnt + jnp.where(v != 0, one, zero)
    denom = jnp.maximum(cnt, 1.0)
    a_v[pl.ds(r16, 16)] = 1.0 / denom
    b_v[pl.ds(r16, 16)] = (float(SEQ) - cnt) / denom
    return _
  lax.fori_loop(0, NG, _count, None)

  for cp in tx_copies:
    cp.wait()

  r0_lo = row0_v[0, pl.ds(0, 16)]
  r0_hi = row0_v[0, pl.ds(16, 16)]
  lanes = lax.iota(jnp.int32, 16)

  def _scale(g, _):
    r16 = pl.multiple_of(g * 16, 16)
    rows = r16 + lanes
    avec = a_v[pl.ds(r16, 16)]
    bvec = b_v[pl.ds(r16, 16)]
    for cc in range(EMB):
      col = jnp.full((16,), cc, jnp.int32)
      accv = plsc.load_gather(acc_v, [rows, col])
      r0c = r0_lo[cc] if cc < 16 else r0_hi[cc - 16]
      txtT_v[cc, pl.ds(r16, 16)] = accv * avec - r0c * bvec
    return _
  lax.fori_loop(0, NG, _scale, None)

  for cp in id_copies:
    cp.wait()

  pltpu.sync_copy(idcols_v, outT_hbm.at[pl.ds(0, EMB), pl.ds(base, RPW)])
  pltpu.sync_copy(txtT_v, outT_hbm.at[pl.ds(EMB, EMB), pl.ds(base, RPW)])


@functools.partial(jax.jit, static_argnums=())
def _run(jnrl_id_idx, tokT, idtabT_flat, text_table):
  mesh = plsc.VectorSubcoreMesh(core_axis_name="c", subcore_axis_name="s")
  f = pl.kernel(
      _sc_body,
      out_type=jax.ShapeDtypeStruct((2 * EMB, B), jnp.float32),
      mesh=mesh,
      compiler_params=pltpu.CompilerParams(
          use_tc_tiling_on_sc=False, needs_layout_passes=False),
      scratch_types=[
          pltpu.VMEM((SEQ, RPW), jnp.int32),
          pltpu.VMEM((RPW,), jnp.int32),
          pltpu.VMEM((EMB, RPW), jnp.int32),
          pltpu.VMEM((EMB, RPW), jnp.float32),
          pltpu.VMEM((RPW, EMB), jnp.float32),
          pltpu.VMEM((EMB, RPW), jnp.float32),
          pltpu.VMEM((RPW,), jnp.float32),
          pltpu.VMEM((RPW,), jnp.float32),
          pltpu.VMEM((1, EMB), jnp.float32),
          pltpu.SemaphoreType.DMA,
          pltpu.SemaphoreType.DMA,
      ],
  )
  outT = f(jnrl_id_idx, tokT, idtabT_flat, text_table)
  return jnp.transpose(outT)


def kernel(jnrl_id_idx, text_token_ids, id_table, text_table):
  tokT = jnp.transpose(text_token_ids)
  idtabT_flat = jnp.transpose(id_table).reshape(-1)
  return _run(jnrl_id_idx, tokT, idtabT_flat, text_table)


# speedup vs baseline: 1.4548x; 1.0785x over previous
import functools

import jax
import jax.numpy as jnp
from jax import lax
from jax.experimental import pallas as pl
from jax.experimental.pallas import tpu as pltpu
from jax.experimental.pallas import tpu_sc as plsc

B = 16384
ID_V = 100001
EMB = 32
SEQ = 20
NW = 32
RPW = B // NW
CH = 128
NCH = RPW // CH
NG = RPW // 16


def _sc_body(idx_hbm, tokT_hbm, idtabT_hbm, txttab_hbm, outT_hbm,
             toks_v, ididx_v, gidx_v, idcols_v, acc_v, txtT_v, a_v, b_v,
             row0_v, sem_id, sem_tx):
  c = lax.axis_index("c")
  s = lax.axis_index("s")
  wid = s * 2 + c
  base = wid * RPW

  def _zero(r, _):
    z = jnp.zeros((16,), jnp.float32)
    acc_v[r, pl.ds(0, 16)] = z
    acc_v[r, pl.ds(16, 16)] = z
    return _
  lax.fori_loop(0, RPW, _zero, None)

  pltpu.sync_copy(tokT_hbm.at[:, pl.ds(base, RPW)], toks_v)
  pltpu.sync_copy(idx_hbm.at[pl.ds(base, RPW)], ididx_v)
  pltpu.sync_copy(txttab_hbm.at[pl.ds(0, 1)], row0_v)

  tx_copies = []
  for j in range(SEQ):
    cp = pltpu.make_async_copy(
        txttab_hbm.at[toks_v.at[j]],
        acc_v,
        sem_tx)
    cp.start(add=True)
    tx_copies.append(cp)

  def _gidx(g, _):
    r16 = pl.multiple_of(g * 16, 16)
    iv = ididx_v[pl.ds(r16, 16)]
    for cc in range(EMB):
      gidx_v[cc, pl.ds(r16, 16)] = iv + (cc * ID_V)
    return _
  lax.fori_loop(0, NG, _gidx, None)

  id_copies = []
  for cc in range(EMB):
    cp = pltpu.make_async_copy(
        idtabT_hbm.at[gidx_v.at[cc]],
        idcols_v.at[cc],
        sem_id)
    cp.start()
    id_copies.append(cp)

  one = jnp.ones((16,), jnp.float32)
  zero = jnp.zeros((16,), jnp.float32)

  def _count(g, _):
    r16 = pl.multiple_of(g * 16, 16)
    cnt = jnp.zeros((16,), jnp.float32)
    for j in range(SEQ):
      v = toks_v[j, pl.ds(r16, 16)]
      cnt = cnt + jnp.where(v != 0, one, zero)
    denom = jnp.maximum(cnt, 1.0)
    a_v[pl.ds(r16, 16)] = 1.0 / denom
    b_v[pl.ds(r16, 16)] = (float(SEQ) - cnt) / denom
    return _
  lax.fori_loop(0, NG, _count, None)

  for cp in tx_copies:
    cp.wait()

  r0_lo = row0_v[0, pl.ds(0, 16)]
  r0_hi = row0_v[0, pl.ds(16, 16)]
  lanes = lax.iota(jnp.int32, 16)

  def _scale(g, _):
    r16 = pl.multiple_of(g * 16, 16)
    rows = r16 + lanes
    avec = a_v[pl.ds(r16, 16)]
    bvec = b_v[pl.ds(r16, 16)]
    for cc in range(EMB):
      col = jnp.full((16,), cc, jnp.int32)
      accv = plsc.load_gather(acc_v, [rows, col])
      r0c = r0_lo[cc] if cc < 16 else r0_hi[cc - 16]
      txtT_v[cc, pl.ds(r16, 16)] = accv * avec - r0c * bvec
    return _
  lax.fori_loop(0, NG, _scale, None)

  for cp in id_copies:
    cp.wait()

  pltpu.sync_copy(idcols_v, outT_hbm.at[pl.ds(0, EMB), pl.ds(base, RPW)])
  pltpu.sync_copy(txtT_v, outT_hbm.at[pl.ds(EMB, EMB), pl.ds(base, RPW)])


@functools.partial(jax.jit, static_argnums=())
def _run(jnrl_id_idx, tokT, idtabT_flat, text_table):
  mesh = plsc.VectorSubcoreMesh(core_axis_name="c", subcore_axis_name="s")
  f = pl.kernel(
      _sc_body,
      out_type=jax.ShapeDtypeStruct((2 * EMB, B), jnp.float32),
      mesh=mesh,
      compiler_params=pltpu.CompilerParams(
          use_tc_tiling_on_sc=False, needs_layout_passes=False),
      scratch_types=[
          pltpu.VMEM((SEQ, RPW), jnp.int32),
          pltpu.VMEM((RPW,), jnp.int32),
          pltpu.VMEM((EMB, RPW), jnp.int32),
          pltpu.VMEM((EMB, RPW), jnp.float32),
          pltpu.VMEM((RPW, EMB), jnp.float32),
          pltpu.VMEM((EMB, RPW), jnp.float32),
          pltpu.VMEM((RPW,), jnp.float32),
          pltpu.VMEM((RPW,), jnp.float32),
          pltpu.VMEM((1, EMB), jnp.float32),
          pltpu.SemaphoreType.DMA,
          pltpu.SemaphoreType.DMA,
      ],
  )
  outT = f(jnrl_id_idx, tokT, idtabT_flat, text_table)
  return jnp.transpose(outT)


def kernel(jnrl_id_idx, text_token_ids, id_table, text_table):
  tokT = jnp.transpose(text_token_ids)
  idtabT_flat = jnp.transpose(id_table).reshape(-1)
  return _run(jnrl_id_idx, tokT, idtabT_flat, text_table)
